# baseline (speedup 1.0000x reference)
import jax
import jax.numpy as jnp
from jax.experimental import pallas as pl
from jax.experimental.pallas import tpu as pltpu

_D = 128
_CH = 2000
_NBUF = 48
_LAG = 40

_CHUNKS = []
for _i in range(25):
    _CHUNKS.append(("a", 2 * _i))
    _CHUNKS.append(("f", 2 * _i))
    _CHUNKS.append(("a", 2 * _i + 1))
    _CHUNKS.append(("f", 2 * _i + 1))
    _CHUNKS.append(("x", _i))
_TOTAL = len(_CHUNKS)


def _copy_body(x_h, a_h, f_h, ao_h, fo_h, xo_h, *scr):
    bufs = scr[:_NBUF]
    rs = scr[_NBUF:2 * _NBUF]
    ws = scr[2 * _NBUF:3 * _NBUF]
    src = {"a": a_h, "f": f_h, "x": x_h}
    dst = {"a": ao_h, "f": fo_h, "x": xo_h}

    def rd(i):
        arr, c = _CHUNKS[i]
        b = i % _NBUF
        return pltpu.make_async_copy(
            src[arr].at[pl.ds(c * _CH, _CH)], bufs[b], rs[b])

    def wr(i):
        arr, c = _CHUNKS[i]
        b = i % _NBUF
        return pltpu.make_async_copy(
            bufs[b], dst[arr].at[pl.ds(c * _CH, _CH)], ws[b])

    for t in range(_TOTAL + _LAG):
        if t < _TOTAL:
            if t >= _NBUF:
                wr(t - _NBUF).wait()
            rd(t).start()
        j = t - _LAG
        if 0 <= j < _TOTAL:
            rd(j).wait()
            wr(j).start()
    for j in range(max(0, _TOTAL - _NBUF), _TOTAL):
        wr(j).wait()


def kernel(x, author_embed, field_embed):
    out = pl.pallas_call(
        _copy_body,
        in_specs=[
            pl.BlockSpec(memory_space=pl.ANY),
            pl.BlockSpec(memory_space=pl.ANY),
            pl.BlockSpec(memory_space=pl.ANY),
        ],
        out_specs=[
            pl.BlockSpec(memory_space=pl.ANY),
            pl.BlockSpec(memory_space=pl.ANY),
            pl.BlockSpec(memory_space=pl.ANY),
        ],
        out_shape=[
            jax.ShapeDtypeStruct(author_embed.shape, author_embed.dtype),
            jax.ShapeDtypeStruct(field_embed.shape, field_embed.dtype),
            jax.ShapeDtypeStruct(x.shape, x.dtype),
        ],
        scratch_shapes=(
            [pltpu.VMEM((_CH, _D), jnp.float32) for _ in range(_NBUF)]
            + [pltpu.SemaphoreType.DMA for _ in range(2 * _NBUF)]
        ),
    )(x, author_embed, field_embed)
    return (out[0], out[1], out[2])


# speedup vs baseline: 1.0101x; 1.0101x over previous
import jax
import jax.numpy as jnp
from jax.experimental import pallas as pl
from jax.experimental.pallas import tpu as pltpu

_D = 128
_CH = 10000
_NBUF = 6
_LAG = 3

_CHUNKS = []
for _i in range(5):
    _CHUNKS.append(("a", 2 * _i))
    _CHUNKS.append(("f", 2 * _i))
    _CHUNKS.append(("a", 2 * _i + 1))
    _CHUNKS.append(("f", 2 * _i + 1))
    _CHUNKS.append(("x", _i))
_TOTAL = len(_CHUNKS)


def _copy_body(x_h, a_h, f_h, ao_h, fo_h, xo_h, *scr):
    bufs = scr[:_NBUF]
    rs = scr[_NBUF:2 * _NBUF]
    ws = scr[2 * _NBUF:3 * _NBUF]
    src = {"a": a_h, "f": f_h, "x": x_h}
    dst = {"a": ao_h, "f": fo_h, "x": xo_h}

    def rd(i):
        arr, c = _CHUNKS[i]
        b = i % _NBUF
        return pltpu.make_async_copy(
            src[arr].at[pl.ds(c * _CH, _CH)], bufs[b], rs[b])

    def wr(i):
        arr, c = _CHUNKS[i]
        b = i % _NBUF
        return pltpu.make_async_copy(
            bufs[b], dst[arr].at[pl.ds(c * _CH, _CH)], ws[b])

    for t in range(_TOTAL + _LAG):
        if t < _TOTAL:
            if t >= _NBUF:
                wr(t - _NBUF).wait()
            rd(t).start()
        j = t - _LAG
        if 0 <= j < _TOTAL:
            rd(j).wait()
            wr(j).start()
    for j in range(max(0, _TOTAL - _NBUF), _TOTAL):
        wr(j).wait()


def kernel(x, author_embed, field_embed):
    out = pl.pallas_call(
        _copy_body,
        in_specs=[
            pl.BlockSpec(memory_space=pl.ANY),
            pl.BlockSpec(memory_space=pl.ANY),
            pl.BlockSpec(memory_space=pl.ANY),
        ],
        out_specs=[
            pl.BlockSpec(memory_space=pl.ANY),
            pl.BlockSpec(memory_space=pl.ANY),
            pl.BlockSpec(memory_space=pl.ANY),
        ],
        out_shape=[
            jax.ShapeDtypeStruct(author_embed.shape, author_embed.dtype),
            jax.ShapeDtypeStruct(field_embed.shape, field_embed.dtype),
            jax.ShapeDtypeStruct(x.shape, x.dtype),
        ],
        scratch_shapes=(
            [pltpu.VMEM((_CH, _D), jnp.float32) for _ in range(_NBUF)]
            + [pltpu.SemaphoreType.DMA for _ in range(2 * _NBUF)]
        ),
    )(x, author_embed, field_embed)
    return (out[0], out[1], out[2])


# speedup vs baseline: 1.0101x; 1.0000x over previous
import jax
import jax.numpy as jnp
from jax.experimental import pallas as pl
from jax.experimental.pallas import tpu as pltpu

_D = 128
_CH = 10000
_NBUF = 10
_LAG = 6

_CHUNKS = []
for _i in range(5):
    _CHUNKS.append(("a", 2 * _i))
    _CHUNKS.append(("f", 2 * _i))
    _CHUNKS.append(("a", 2 * _i + 1))
    _CHUNKS.append(("f", 2 * _i + 1))
    _CHUNKS.append(("x", _i))
_TOTAL = len(_CHUNKS)


def _copy_body(x_h, a_h, f_h, ao_h, fo_h, xo_h, *scr):
    bufs = scr[:_NBUF]
    rs = scr[_NBUF:2 * _NBUF]
    ws = scr[2 * _NBUF:3 * _NBUF]
    src = {"a": a_h, "f": f_h, "x": x_h}
    dst = {"a": ao_h, "f": fo_h, "x": xo_h}

    def rd(i):
        arr, c = _CHUNKS[i]
        b = i % _NBUF
        return pltpu.make_async_copy(
            src[arr].at[pl.ds(c * _CH, _CH)], bufs[b], rs[b])

    def wr(i):
        arr, c = _CHUNKS[i]
        b = i % _NBUF
        return pltpu.make_async_copy(
            bufs[b], dst[arr].at[pl.ds(c * _CH, _CH)], ws[b])

    for t in range(_TOTAL + _LAG):
        if t < _TOTAL:
            if t >= _NBUF:
                wr(t - _NBUF).wait()
            rd(t).start()
        j = t - _LAG
        if 0 <= j < _TOTAL:
            rd(j).wait()
            wr(j).start()
    for j in range(max(0, _TOTAL - _NBUF), _TOTAL):
        wr(j).wait()


def kernel(x, author_embed, field_embed):
    out = pl.pallas_call(
        _copy_body,
        in_specs=[
            pl.BlockSpec(memory_space=pl.ANY),
            pl.BlockSpec(memory_space=pl.ANY),
            pl.BlockSpec(memory_space=pl.ANY),
        ],
        out_specs=[
            pl.BlockSpec(memory_space=pl.ANY),
            pl.BlockSpec(memory_space=pl.ANY),
            pl.BlockSpec(memory_space=pl.ANY),
        ],
        out_shape=[
            jax.ShapeDtypeStruct(author_embed.shape, author_embed.dtype),
            jax.ShapeDtypeStruct(field_embed.shape, field_embed.dtype),
            jax.ShapeDtypeStruct(x.shape, x.dtype),
        ],
        scratch_shapes=(
            [pltpu.VMEM((_CH, _D), jnp.float32) for _ in range(_NBUF)]
            + [pltpu.SemaphoreType.DMA for _ in range(2 * _NBUF)]
        ),
    )(x, author_embed, field_embed)
    return (out[0], out[1], out[2])
